# Initial kernel scaffold; baseline (speedup 1.0000x reference)
import jax
import jax.numpy as jnp
from jax.experimental import pallas as pl


def kernel(s, v, edge_index_local, d_local, vec_local, edge_index_global, d_global, vec_global, params):
    raise NotImplementedError("write your pallas kernel here")


# speedup vs baseline: 19.3904x; 19.3904x over previous
import functools

import jax
import jax.numpy as jnp
from jax import lax
from jax.experimental import pallas as pl
from jax.experimental.pallas import tpu as pltpu
from jax.experimental.pallas import tpu_sc as plsc

SDIM = 64
VDIM = 16
V3 = 3 * VDIM
RBF_DIM = 64
CUTOFF = 5.0
NUM_LAYERS = 5

_NC = 2
_NS = 16
_NW = _NC * _NS

_NROW = 25600
_NGARB = 64
_NTAB = _NROW + _NGARB
_NPAD = _NC * _NROW


def _node_block(n):
    for b in (2000, 1000, 500, 250, 200, 100, 50, 25, 10, 5, 1):
        if n % b == 0 and (b % 8 == 0 or b == n):
            return b
    return 1


def _edge_block(e):
    for b in (2000, 1000, 500, 250, 200, 100, 50, 25, 10, 5, 1):
        if e % b == 0 and (b % 8 == 0 or b == e):
            return b
    return 1


def _prep_body(s_ref, v_ref, gam_ref, bet_ref, w1a_ref, w1b_ref, b1_ref,
               sln_ref, vln_ref, a_ref, b_ref):
    s = s_ref[...]
    mu = jnp.mean(s, axis=-1, keepdims=True)
    var = jnp.mean((s - mu) ** 2, axis=-1, keepdims=True)
    sln = (s - mu) / jnp.sqrt(var + 1e-5) * gam_ref[...] + bet_ref[...]
    vv = v_ref[...]
    vn = jnp.sqrt(jnp.sum(vv * vv, axis=-1, keepdims=True) / VDIM + 1e-5)
    sln_ref[...] = sln
    vln_ref[...] = vv / vn
    a_ref[...] = jnp.dot(sln, w1a_ref[...],
                         preferred_element_type=jnp.float32) + b1_ref[...]
    b_ref[...] = jnp.dot(sln, w1b_ref[...], preferred_element_type=jnp.float32)


def _node_prep(s, v2, gam, bet, w1a, w1b, b1):
    n = s.shape[0]
    bn = _node_block(n)
    grid = n // bn
    row = lambda i: (i, 0)
    full = lambda i: (0, 0)
    out = pl.pallas_call(
        _prep_body,
        grid=(grid,),
        in_specs=[
            pl.BlockSpec((bn, SDIM), row),
            pl.BlockSpec((bn, V3), row),
            pl.BlockSpec((1, SDIM), full),
            pl.BlockSpec((1, SDIM), full),
            pl.BlockSpec((SDIM, SDIM), full),
            pl.BlockSpec((SDIM, SDIM), full),
            pl.BlockSpec((1, SDIM), full),
        ],
        out_specs=[
            pl.BlockSpec((bn, SDIM), row),
            pl.BlockSpec((bn, V3), row),
            pl.BlockSpec((bn, SDIM), row),
            pl.BlockSpec((bn, SDIM), row),
        ],
        out_shape=[
            jax.ShapeDtypeStruct((n, SDIM), jnp.float32),
            jax.ShapeDtypeStruct((n, V3), jnp.float32),
            jax.ShapeDtypeStruct((n, SDIM), jnp.float32),
            jax.ShapeDtypeStruct((n, SDIM), jnp.float32),
        ],
    )(s, v2, gam, bet, w1a, w1b, b1)
    return out


def _edge_body(use_rbf, has_v, refs):
    i = 0
    asrc = refs[i][...]; i += 1
    bdst = refs[i][...]; i += 1
    d = refs[i][...]; i += 1
    vec = refs[i][...]; i += 1
    if has_v:
        vsrc = refs[i][...]; i += 1
    if use_rbf:
        wfold = refs[i][...]; i += 1
    else:
        w_d = refs[i][...]; i += 1
        b_d = refs[i][...]; i += 1
        w1c = refs[i][...]; i += 1
    w2s = refs[i][...]; i += 1
    b2s = refs[i][...]; i += 1
    w2g = refs[i][...]; i += 1
    b2g = refs[i][...]; i += 1
    smsg_ref = refs[i]; i += 1
    vmsg_ref = refs[i]; i += 1

    if use_rbf:
        mu = lax.broadcasted_iota(jnp.int32, (1, RBF_DIM), 1).astype(
            jnp.float32) * (CUTOFF / (RBF_DIM - 1))
        gamma = 1.0 / ((CUTOFF / RBF_DIM) ** 2)
        rbf = jnp.exp(-gamma * (d - mu) ** 2)
        c = 0.5 * (jnp.cos(jnp.pi * jnp.clip(d, 0.0, CUTOFF) / CUTOFF) + 1.0)
        eterm = c * jnp.dot(rbf, wfold, preferred_element_type=jnp.float32)
    else:
        e = jnp.tanh(d * w_d + b_d)
        eterm = jnp.dot(e, w1c, preferred_element_type=jnp.float32)
    h = asrc + bdst + eterm
    h = h * jax.nn.sigmoid(h)
    smsg_ref[...] = jnp.dot(h, w2s, preferred_element_type=jnp.float32) + b2s
    g = jnp.dot(h, w2g, preferred_element_type=jnp.float32) + b2g
    gv = g[:, :VDIM]
    gd = g[:, VDIM:2 * VDIM]
    ss = jnp.sum(vec * vec, axis=-1, keepdims=True)
    dv = vec / (jnp.sqrt(ss) + 1e-8)
    vmsg = jnp.concatenate(
        [dv[:, 0:1] * gd, dv[:, 1:2] * gd, dv[:, 2:3] * gd], axis=-1)
    if has_v:
        gv3 = jnp.concatenate([gv, gv, gv], axis=-1)
        vmsg = vmsg + vsrc * gv3
    vmsg_ref[...] = vmsg


def _edge_mlp(use_rbf, has_v, asrc, bdst, d1, vec, vsrc, wts):
    e = asrc.shape[0]
    be = _edge_block(e)
    grid = e // be
    row = lambda i: (i, 0)
    full = lambda i: (0, 0)
    in_specs = [
        pl.BlockSpec((be, SDIM), row),
        pl.BlockSpec((be, SDIM), row),
        pl.BlockSpec((be, 1), row),
        pl.BlockSpec((be, 3), row),
    ]
    args = [asrc, bdst, d1, vec]
    if has_v:
        in_specs.append(pl.BlockSpec((be, V3), row))
        args.append(vsrc)
    for w in wts:
        in_specs.append(pl.BlockSpec(w.shape, full))
        args.append(w)
    body = lambda *refs: _edge_body(use_rbf, has_v, refs)
    return pl.pallas_call(
        body,
        grid=(grid,),
        in_specs=in_specs,
        out_specs=[
            pl.BlockSpec((be, SDIM), row),
            pl.BlockSpec((be, V3), row),
        ],
        out_shape=[
            jax.ShapeDtypeStruct((e, SDIM), jnp.float32),
            jax.ShapeDtypeStruct((e, V3), jnp.float32),
        ],
    )(*args)


def _update_body(use_mlp, refs):
    i = 0
    sln = refs[i][...]; i += 1
    vln = refs[i][...]; i += 1
    sagg = refs[i][...]; i += 1
    vagg = refs[i][...]; i += 1
    cnt = refs[i][...]; i += 1
    if use_mlp:
        wu1 = refs[i][...]; i += 1
        bu1 = refs[i][...]; i += 1
        wu2 = refs[i][...]; i += 1
        bu2 = refs[i][...]; i += 1
    s_ref = refs[i]; i += 1
    v_ref = refs[i]; i += 1
    s_new = sln + sagg
    inv = 1.0 / jnp.maximum(cnt, 1.0)
    v_ref[...] = vln + vagg * inv
    if use_mlp:
        h = jnp.dot(s_new, wu1, preferred_element_type=jnp.float32) + bu1
        h = h * jax.nn.sigmoid(h)
        s_new = s_new + jnp.dot(h, wu2, preferred_element_type=jnp.float32) + bu2
    s_ref[...] = s_new


def _node_update(use_mlp, sln, vln, sagg, vagg, cnt, wts):
    n = sln.shape[0]
    bn = _node_block(n)
    grid = n // bn
    row = lambda i: (i, 0)
    full = lambda i: (0, 0)
    in_specs = [
        pl.BlockSpec((bn, SDIM), row),
        pl.BlockSpec((bn, V3), row),
        pl.BlockSpec((bn, SDIM), row),
        pl.BlockSpec((bn, V3), row),
        pl.BlockSpec((bn, 1), row),
    ]
    args = [sln, vln, sagg, vagg, cnt]
    for w in wts:
        in_specs.append(pl.BlockSpec(w.shape, full))
        args.append(w)
    body = lambda *refs: _update_body(use_mlp, refs)
    return pl.pallas_call(
        body,
        grid=(grid,),
        in_specs=in_specs,
        out_specs=[
            pl.BlockSpec((bn, SDIM), row),
            pl.BlockSpec((bn, V3), row),
        ],
        out_shape=[
            jax.ShapeDtypeStruct((n, SDIM), jnp.float32),
            jax.ShapeDtypeStruct((n, V3), jnp.float32),
        ],
    )(*args)


def _mesh():
    return plsc.VectorSubcoreMesh(core_axis_name="c", subcore_axis_name="s")


_SC_PARAMS = pltpu.CompilerParams(use_tc_tiling_on_sc=False)


def _gather_chunk(table_hbm, idx_hbm, out_hbm, idx_v, rows_v, sem, off, sizes):
    total = sum(sizes)
    pltpu.sync_copy(idx_hbm.at[pl.ds(off, total)], idx_v.at[pl.ds(0, total)])
    descs = []
    pos = 0
    for bs in sizes:
        descs.append(pltpu.async_copy(
            table_hbm.at[idx_v.at[pl.ds(pos, bs)]],
            rows_v.at[pl.ds(pos, bs)], sem))
        pos += bs
    for dsc in descs:
        dsc.wait()
    pltpu.sync_copy(rows_v.at[pl.ds(0, total)], out_hbm.at[pl.ds(off, total)])


def _sc_gather(table, idx):
    n, dd = table.shape
    e = idx.shape[0]
    per_w = e // _NW
    nch = per_w // 1024
    tail = per_w - nch * 1024
    tail_sizes = [128] * (tail // 128) + ([tail % 128] if tail % 128 else [])

    @functools.partial(
        pl.kernel, mesh=_mesh(),
        out_type=jax.ShapeDtypeStruct((e, dd), jnp.float32),
        scratch_types=[
            pltpu.VMEM((1024,), jnp.int32),
            pltpu.VMEM((1024, dd), jnp.float32),
            pltpu.SemaphoreType.DMA,
        ],
        compiler_params=_SC_PARAMS,
        name="sc_gather%d" % dd)
    def k(table_hbm, idx_hbm, out_hbm, idx_v, rows_v, sem):
        wid = lax.axis_index("s") * _NC + lax.axis_index("c")
        base = wid * per_w

        def body(t, carry):
            _gather_chunk(table_hbm, idx_hbm, out_hbm, idx_v, rows_v, sem,
                          base + t * 1024, [128] * 8)
            return carry
        lax.fori_loop(0, nch, body, 0)
        if tail:
            _gather_chunk(table_hbm, idx_hbm, out_hbm, idx_v, rows_v, sem,
                          base + nch * 1024, tail_sizes)

    return k(table, idx)


def _scatter_chunk(msg_hbm, dst_hbm, table, idxr_v, idx2_v, msg_v, nbase,
                   off, ngroups, nbatch):
    total = ngroups * 16
    pltpu.sync_copy(dst_hbm.at[pl.ds(off, total)], idxr_v.at[pl.ds(0, total)])
    pltpu.sync_copy(msg_hbm.at[pl.ds(off, total)], msg_v.at[pl.ds(0, total)])
    for g in range(ngroups):
        raw = idxr_v[pl.ds(g * 16, 16)]
        li = raw - nbase
        oob = (li < 0) | (li >= _NROW)
        li = jnp.where(oob, _NROW + (raw & (_NGARB - 1)), li)
        idx2_v[g // 8, pl.ds((g % 8) * 16, 16)] = li
    garb = _NROW + jnp.arange(16, dtype=jnp.int32)
    for g in range(ngroups, nbatch * 8):
        idx2_v[g // 8, pl.ds((g % 8) * 16, 16)] = garb
    for j in range(nbatch):
        pltpu.sync_copy(msg_v.at[pl.ds(j * 128, 128)],
                        table.at[idx2_v.at[j]], add=True)


def _sc_scatter(msg, dst):
    e, dd = msg.shape
    per_w = e // _NS
    chunk = 256
    nch = per_w // chunk
    tail = per_w - nch * chunk
    tail_groups = tail // 16
    tail_batch = (tail + 127) // 128
    zrows = _NROW // _NS
    zeros = jnp.zeros((zrows, dd), jnp.float32)

    @functools.partial(
        pl.kernel, mesh=_mesh(),
        out_type=jax.ShapeDtypeStruct((_NPAD, dd), jnp.float32),
        scratch_types=[
            pltpu.VMEM_SHARED((_NTAB, dd), jnp.float32),
            pltpu.VMEM((chunk,), jnp.int32),
            pltpu.VMEM((chunk // 128, 128), jnp.int32),
            pltpu.VMEM((chunk, dd), jnp.float32),
        ],
        compiler_params=_SC_PARAMS,
        name="sc_scatter%d" % dd)
    def k(msg_hbm, dst_hbm, zeros_hbm, out_hbm, table, idxr_v, idx2_v, msg_v):
        cid = lax.axis_index("c")
        sid = lax.axis_index("s")
        nbase = cid * _NROW
        pltpu.sync_copy(zeros_hbm, table.at[pl.ds(sid * zrows, zrows)])
        plsc.subcore_barrier()
        ebase = sid * per_w

        def body(t, carry):
            _scatter_chunk(msg_hbm, dst_hbm, table, idxr_v, idx2_v, msg_v,
                           nbase, ebase + t * chunk, chunk // 16, chunk // 128)
            return carry
        lax.fori_loop(0, nch, body, 0)
        if tail:
            _scatter_chunk(msg_hbm, dst_hbm, table, idxr_v, idx2_v, msg_v,
                           nbase, ebase + nch * chunk, tail_groups, tail_batch)
        plsc.subcore_barrier()
        pltpu.sync_copy(table.at[pl.ds(sid * zrows, zrows)],
                        out_hbm.at[pl.ds(nbase + sid * zrows, zrows)])

    return k(msg, dst, zeros)


def _sc_count(dst):
    e = dst.shape[0]
    per_w = e // _NS
    nch = per_w // 1024
    tail = per_w - nch * 1024
    tail_groups = tail // 16
    tail_batch = (tail + 127) // 128
    zrows = _NROW // _NS
    zeros = jnp.zeros((zrows,), jnp.float32)

    @functools.partial(
        pl.kernel, mesh=_mesh(),
        out_type=jax.ShapeDtypeStruct((_NPAD,), jnp.float32),
        scratch_types=[
            pltpu.VMEM_SHARED((_NTAB,), jnp.float32),
            pltpu.VMEM((1024,), jnp.int32),
            pltpu.VMEM((8, 128), jnp.int32),
            pltpu.VMEM((128,), jnp.float32),
        ],
        compiler_params=_SC_PARAMS,
        name="sc_count")
    def k(dst_hbm, zeros_hbm, out_hbm, table, idxr_v, idx2_v, ones_v):
        cid = lax.axis_index("c")
        sid = lax.axis_index("s")
        nbase = cid * _NROW
        pltpu.sync_copy(zeros_hbm, table.at[pl.ds(sid * zrows, zrows)])
        for i in range(8):
            ones_v[pl.ds(i * 16, 16)] = jnp.ones((16,), jnp.float32)
        plsc.subcore_barrier()
        ebase = sid * per_w

        def chunk(off, ngroups, nbatch):
            total = ngroups * 16
            pltpu.sync_copy(dst_hbm.at[pl.ds(off, total)],
                            idxr_v.at[pl.ds(0, total)])
            for g in range(ngroups):
                raw = idxr_v[pl.ds(g * 16, 16)]
                li = raw - nbase
                oob = (li < 0) | (li >= _NROW)
                li = jnp.where(oob, _NROW + (raw & (_NGARB - 1)), li)
                idx2_v[g // 8, pl.ds((g % 8) * 16, 16)] = li
            garb = _NROW + jnp.arange(16, dtype=jnp.int32)
            for g in range(ngroups, nbatch * 8):
                idx2_v[g // 8, pl.ds((g % 8) * 16, 16)] = garb
            for j in range(nbatch):
                pltpu.sync_copy(ones_v, table.at[idx2_v.at[j]], add=True)

        def body(t, carry):
            chunk(ebase + t * 1024, 64, 8)
            return carry
        lax.fori_loop(0, nch, body, 0)
        if tail:
            chunk(ebase + nch * 1024, tail_groups, tail_batch)
        plsc.subcore_barrier()
        pltpu.sync_copy(table.at[pl.ds(sid * zrows, zrows)],
                        out_hbm.at[pl.ds(nbase + sid * zrows, zrows)])

    return k(dst, zeros)


_gather_impl = _sc_gather
_scatter_impl = _sc_scatter
_count_impl = _sc_count


def kernel(s, v, edge_index_local, d_local, vec_local, edge_index_global,
           d_global, vec_global, params):
    n = s.shape[0]
    v2 = v.reshape(n, V3)
    src_l, dst_l = edge_index_local[0], edge_index_local[1]
    src_g, dst_g = edge_index_global[0], edge_index_global[1]
    d1_l = d_local[:, None]
    d1_g = d_global[:, None]

    cnt_l = _count_impl(dst_l)[:, None]
    cnt_g = _count_impl(dst_g)[:, None]

    for i in range(NUM_LAYERS):
        p = params["layer%d" % i]
        use_global = i == NUM_LAYERS - 2
        use_rbf = not use_global
        has_v = i > 0
        use_mlp = i < NUM_LAYERS - 1
        src, dst = (src_g, dst_g) if use_global else (src_l, dst_l)
        d1 = d1_g if use_global else d1_l
        vec = vec_global if use_global else vec_local
        cnt = cnt_g if use_global else cnt_l

        w1 = p["w1"]
        w1a, w1b, w1c = w1[:SDIM], w1[SDIM:2 * SDIM], w1[2 * SDIM:]
        w2 = p["w2"]
        w2s, w2g = w2[:, :SDIM], w2[:, SDIM:]
        b2 = p["b2"]
        b2s, b2g = b2[None, :SDIM], b2[None, SDIM:]

        sln, vln, a, b = _node_prep(
            s, v2, p["ln_gamma"][None], p["ln_beta"][None], w1a, w1b,
            p["b1"][None])

        asrc = _gather_impl(a, src)
        bdst = _gather_impl(b, dst)
        vsrc = _gather_impl(vln, src) if has_v else None

        if use_rbf:
            wts = [jnp.dot(p["w_rbf"], w1c), w2s, b2s, w2g, b2g]
        else:
            wts = [p["w_d"], p["b_d"][None], w1c, w2s, b2s, w2g, b2g]
        smsg, vmsg = _edge_mlp(use_rbf, has_v, asrc, bdst, d1, vec, vsrc, wts)

        sagg = _scatter_impl(smsg, dst)
        vagg = _scatter_impl(vmsg, dst)

        uwts = [p["wu1"], p["bu1"][None], p["wu2"], p["bu2"][None]] \
            if use_mlp else []
        s, v2 = _node_update(use_mlp, sln, vln, sagg[:n], vagg[:n], cnt[:n],
                             uwts)

    return s, v2.reshape(n, 3, VDIM)


# speedup vs baseline: 23.3669x; 1.2051x over previous
import functools

import jax
import jax.numpy as jnp
from jax import lax
from jax.experimental import pallas as pl
from jax.experimental.pallas import tpu as pltpu
from jax.experimental.pallas import tpu_sc as plsc

SDIM = 64
VDIM = 16
V3 = 3 * VDIM
RBF_DIM = 64
CUTOFF = 5.0
NUM_LAYERS = 5

_NC = 2
_NS = 16
_NW = _NC * _NS

_NROW = 25600
_NGARB = 64
_NTAB = _NROW + _NGARB
_NPAD = _NC * _NROW


def _node_block(n):
    for b in (2000, 1000, 500, 250, 200, 100, 50, 25, 10, 5, 1):
        if n % b == 0 and (b % 8 == 0 or b == n):
            return b
    return 1


def _edge_block(e):
    for b in (4000, 2000, 1000, 500, 250, 200, 100, 50, 25, 10, 5, 1):
        if e % b == 0 and (b % 8 == 0 or b == e):
            return b
    return 1


def _prep_body(s_ref, v_ref, gam_ref, bet_ref, w1a_ref, w1b_ref, b1_ref,
               sln_ref, vln_ref, a_ref, b_ref):
    s = s_ref[...]
    mu = jnp.mean(s, axis=-1, keepdims=True)
    var = jnp.mean((s - mu) ** 2, axis=-1, keepdims=True)
    sln = (s - mu) / jnp.sqrt(var + 1e-5) * gam_ref[...] + bet_ref[...]
    vv = v_ref[...]
    vn = jnp.sqrt(jnp.sum(vv * vv, axis=-1, keepdims=True) / VDIM + 1e-5)
    sln_ref[...] = sln
    vln_ref[...] = vv / vn
    a_ref[...] = jnp.dot(sln, w1a_ref[...],
                         preferred_element_type=jnp.float32) + b1_ref[...]
    b_ref[...] = jnp.dot(sln, w1b_ref[...], preferred_element_type=jnp.float32)


def _node_prep(s, v2, gam, bet, w1a, w1b, b1):
    n = s.shape[0]
    bn = _node_block(n)
    grid = n // bn
    row = lambda i: (i, 0)
    full = lambda i: (0, 0)
    out = pl.pallas_call(
        _prep_body,
        grid=(grid,),
        in_specs=[
            pl.BlockSpec((bn, SDIM), row),
            pl.BlockSpec((bn, V3), row),
            pl.BlockSpec((1, SDIM), full),
            pl.BlockSpec((1, SDIM), full),
            pl.BlockSpec((SDIM, SDIM), full),
            pl.BlockSpec((SDIM, SDIM), full),
            pl.BlockSpec((1, SDIM), full),
        ],
        out_specs=[
            pl.BlockSpec((bn, SDIM), row),
            pl.BlockSpec((bn, V3), row),
            pl.BlockSpec((bn, SDIM), row),
            pl.BlockSpec((bn, SDIM), row),
        ],
        out_shape=[
            jax.ShapeDtypeStruct((n, SDIM), jnp.float32),
            jax.ShapeDtypeStruct((n, V3), jnp.float32),
            jax.ShapeDtypeStruct((n, SDIM), jnp.float32),
            jax.ShapeDtypeStruct((n, SDIM), jnp.float32),
        ],
        name="tc_node_prep",
    )(s, v2, gam, bet, w1a, w1b, b1)
    return out


def _upprep_body(sln_ref, vln_ref, sagg_ref, vagg_ref, cnt_ref,
                 wu1_ref, bu1_ref, wu2_ref, bu2_ref,
                 gam_ref, bet_ref, w1a_ref, w1b_ref, b1_ref,
                 sln2_ref, vln2_ref, a_ref, b_ref):
    s_new = sln_ref[...] + sagg_ref[...]
    inv = 1.0 / jnp.maximum(cnt_ref[...], 1.0)
    v_new = vln_ref[...] + vagg_ref[...] * inv
    h = jnp.dot(s_new, wu1_ref[...], preferred_element_type=jnp.float32) \
        + bu1_ref[...]
    h = h * jax.nn.sigmoid(h)
    s_new = s_new + jnp.dot(h, wu2_ref[...],
                            preferred_element_type=jnp.float32) + bu2_ref[...]
    mu = jnp.mean(s_new, axis=-1, keepdims=True)
    var = jnp.mean((s_new - mu) ** 2, axis=-1, keepdims=True)
    sln = (s_new - mu) / jnp.sqrt(var + 1e-5) * gam_ref[...] + bet_ref[...]
    vn = jnp.sqrt(jnp.sum(v_new * v_new, axis=-1, keepdims=True) / VDIM + 1e-5)
    sln2_ref[...] = sln
    vln2_ref[...] = v_new / vn
    a_ref[...] = jnp.dot(sln, w1a_ref[...],
                         preferred_element_type=jnp.float32) + b1_ref[...]
    b_ref[...] = jnp.dot(sln, w1b_ref[...], preferred_element_type=jnp.float32)


def _node_update_prep(sln, vln, sagg, vagg, cnt, uwts, pwts):
    n = sln.shape[0]
    bn = _node_block(n)
    grid = n // bn
    row = lambda i: (i, 0)
    full = lambda i: (0, 0)
    in_specs = [
        pl.BlockSpec((bn, SDIM), row),
        pl.BlockSpec((bn, V3), row),
        pl.BlockSpec((bn, SDIM), row),
        pl.BlockSpec((bn, V3), row),
        pl.BlockSpec((bn, 1), row),
    ]
    args = [sln, vln, sagg, vagg, cnt]
    for w in uwts + pwts:
        in_specs.append(pl.BlockSpec(w.shape, full))
        args.append(w)
    return pl.pallas_call(
        _upprep_body,
        grid=(grid,),
        in_specs=in_specs,
        out_specs=[
            pl.BlockSpec((bn, SDIM), row),
            pl.BlockSpec((bn, V3), row),
            pl.BlockSpec((bn, SDIM), row),
            pl.BlockSpec((bn, SDIM), row),
        ],
        out_shape=[
            jax.ShapeDtypeStruct((n, SDIM), jnp.float32),
            jax.ShapeDtypeStruct((n, V3), jnp.float32),
            jax.ShapeDtypeStruct((n, SDIM), jnp.float32),
            jax.ShapeDtypeStruct((n, SDIM), jnp.float32),
        ],
        name="tc_update_prep",
    )(*args)


def _estat_body(use_rbf, refs):
    i = 0
    d = refs[i][...]; i += 1
    vec = refs[i][...]; i += 1
    if not use_rbf:
        w_d = refs[i][...]; i += 1
        b_d = refs[i][...]; i += 1
    x_ref = refs[i]; i += 1
    dir_ref = refs[i]; i += 1
    if use_rbf:
        mu = lax.broadcasted_iota(jnp.int32, (1, RBF_DIM), 1).astype(
            jnp.float32) * (CUTOFF / (RBF_DIM - 1))
        gamma = 1.0 / ((CUTOFF / RBF_DIM) ** 2)
        rbf = jnp.exp(-gamma * (d - mu) ** 2)
        c = 0.5 * (jnp.cos(jnp.pi * jnp.clip(d, 0.0, CUTOFF) / CUTOFF) + 1.0)
        x_ref[...] = rbf * c
    else:
        x_ref[...] = jnp.tanh(d * w_d + b_d)
    ss = jnp.sum(vec * vec, axis=-1, keepdims=True)
    dv = vec / (jnp.sqrt(ss) + 1e-8)
    ones = jnp.ones((1, VDIM), jnp.float32)
    dir_ref[...] = jnp.concatenate(
        [dv[:, 0:1] * ones, dv[:, 1:2] * ones, dv[:, 2:3] * ones], axis=-1)


def _edge_static(use_rbf, d1, vec, wts):
    e = d1.shape[0]
    be = _edge_block(e)
    grid = e // be
    row = lambda i: (i, 0)
    full = lambda i: (0, 0)
    in_specs = [pl.BlockSpec((be, 1), row), pl.BlockSpec((be, 3), row)]
    args = [d1, vec]
    for w in wts:
        in_specs.append(pl.BlockSpec(w.shape, full))
        args.append(w)
    body = lambda *refs: _estat_body(use_rbf, refs)
    return pl.pallas_call(
        body,
        grid=(grid,),
        in_specs=in_specs,
        out_specs=[
            pl.BlockSpec((be, SDIM), row),
            pl.BlockSpec((be, V3), row),
        ],
        out_shape=[
            jax.ShapeDtypeStruct((e, SDIM), jnp.float32),
            jax.ShapeDtypeStruct((e, V3), jnp.float32),
        ],
        name="tc_edge_static",
    )(*args)


def _edge_body(has_v, refs):
    i = 0
    asrc = refs[i][...]; i += 1
    bdst = refs[i][...]; i += 1
    x = refs[i][...]; i += 1
    dir48 = refs[i][...]; i += 1
    if has_v:
        vsrc = refs[i][...]; i += 1
    ww = refs[i][...]; i += 1
    w2s = refs[i][...]; i += 1
    b2s = refs[i][...]; i += 1
    w2gv = refs[i][...]; i += 1
    b2gv = refs[i][...]; i += 1
    w2gd = refs[i][...]; i += 1
    b2gd = refs[i][...]; i += 1
    smsg_ref = refs[i]; i += 1
    vmsg_ref = refs[i]; i += 1

    h = asrc + bdst + jnp.dot(x, ww, preferred_element_type=jnp.float32)
    h = h * jax.nn.sigmoid(h)
    smsg_ref[...] = jnp.dot(h, w2s, preferred_element_type=jnp.float32) + b2s
    gv = jnp.dot(h, w2gv, preferred_element_type=jnp.float32) + b2gv
    gd = jnp.dot(h, w2gd, preferred_element_type=jnp.float32) + b2gd
    vmsg = dir48 * jnp.concatenate([gd, gd, gd], axis=-1)
    if has_v:
        vmsg = vmsg + vsrc * jnp.concatenate([gv, gv, gv], axis=-1)
    vmsg_ref[...] = vmsg


def _edge_mlp(has_v, asrc, bdst, x, dir48, vsrc, wts):
    e = asrc.shape[0]
    be = _edge_block(e)
    grid = e // be
    row = lambda i: (i, 0)
    full = lambda i: (0, 0)
    in_specs = [
        pl.BlockSpec((be, SDIM), row),
        pl.BlockSpec((be, SDIM), row),
        pl.BlockSpec((be, SDIM), row),
        pl.BlockSpec((be, V3), row),
    ]
    args = [asrc, bdst, x, dir48]
    if has_v:
        in_specs.append(pl.BlockSpec((be, V3), row))
        args.append(vsrc)
    for w in wts:
        in_specs.append(pl.BlockSpec(w.shape, full))
        args.append(w)
    body = lambda *refs: _edge_body(has_v, refs)
    return pl.pallas_call(
        body,
        grid=(grid,),
        in_specs=in_specs,
        out_specs=[
            pl.BlockSpec((be, SDIM), row),
            pl.BlockSpec((be, V3), row),
        ],
        out_shape=[
            jax.ShapeDtypeStruct((e, SDIM), jnp.float32),
            jax.ShapeDtypeStruct((e, V3), jnp.float32),
        ],
        name="tc_edge_mlp",
    )(*args)


def _update_body(use_mlp, refs):
    i = 0
    sln = refs[i][...]; i += 1
    vln = refs[i][...]; i += 1
    sagg = refs[i][...]; i += 1
    vagg = refs[i][...]; i += 1
    cnt = refs[i][...]; i += 1
    if use_mlp:
        wu1 = refs[i][...]; i += 1
        bu1 = refs[i][...]; i += 1
        wu2 = refs[i][...]; i += 1
        bu2 = refs[i][...]; i += 1
    s_ref = refs[i]; i += 1
    v_ref = refs[i]; i += 1
    s_new = sln + sagg
    inv = 1.0 / jnp.maximum(cnt, 1.0)
    v_ref[...] = vln + vagg * inv
    if use_mlp:
        h = jnp.dot(s_new, wu1, preferred_element_type=jnp.float32) + bu1
        h = h * jax.nn.sigmoid(h)
        s_new = s_new + jnp.dot(h, wu2, preferred_element_type=jnp.float32) + bu2
    s_ref[...] = s_new


def _node_update(use_mlp, sln, vln, sagg, vagg, cnt, wts):
    n = sln.shape[0]
    bn = _node_block(n)
    grid = n // bn
    row = lambda i: (i, 0)
    full = lambda i: (0, 0)
    in_specs = [
        pl.BlockSpec((bn, SDIM), row),
        pl.BlockSpec((bn, V3), row),
        pl.BlockSpec((bn, SDIM), row),
        pl.BlockSpec((bn, V3), row),
        pl.BlockSpec((bn, 1), row),
    ]
    args = [sln, vln, sagg, vagg, cnt]
    for w in wts:
        in_specs.append(pl.BlockSpec(w.shape, full))
        args.append(w)
    body = lambda *refs: _update_body(use_mlp, refs)
    return pl.pallas_call(
        body,
        grid=(grid,),
        in_specs=in_specs,
        out_specs=[
            pl.BlockSpec((bn, SDIM), row),
            pl.BlockSpec((bn, V3), row),
        ],
        out_shape=[
            jax.ShapeDtypeStruct((n, SDIM), jnp.float32),
            jax.ShapeDtypeStruct((n, V3), jnp.float32),
        ],
        name="tc_node_update",
    )(*args)


def _mesh():
    return plsc.VectorSubcoreMesh(core_axis_name="c", subcore_axis_name="s")


_SC_PARAMS = pltpu.CompilerParams(use_tc_tiling_on_sc=False)


def _gather_chunk(table_hbm, idx_hbm, out_hbm, idx_v, rows_v, sem, off, sizes):
    total = sum(sizes)
    pltpu.sync_copy(idx_hbm.at[pl.ds(off, total)], idx_v.at[pl.ds(0, total)])
    descs = []
    pos = 0
    for bs in sizes:
        descs.append(pltpu.async_copy(
            table_hbm.at[idx_v.at[pl.ds(pos, bs)]],
            rows_v.at[pl.ds(pos, bs)], sem))
        pos += bs
    for dsc in descs:
        dsc.wait()
    pltpu.sync_copy(rows_v.at[pl.ds(0, total)], out_hbm.at[pl.ds(off, total)])


def _sc_gather(table, idx):
    n, dd = table.shape
    e = idx.shape[0]
    per_w = e // _NW
    nch = per_w // 1024
    tail = per_w - nch * 1024
    tail_sizes = [128] * (tail // 128) + ([tail % 128] if tail % 128 else [])

    @functools.partial(
        pl.kernel, mesh=_mesh(),
        out_type=jax.ShapeDtypeStruct((e, dd), jnp.float32),
        scratch_types=[
            pltpu.VMEM((1024,), jnp.int32),
            pltpu.VMEM((1024, dd), jnp.float32),
            pltpu.SemaphoreType.DMA,
        ],
        compiler_params=_SC_PARAMS,
        name="sc_gather%d" % dd)
    def k(table_hbm, idx_hbm, out_hbm, idx_v, rows_v, sem):
        wid = lax.axis_index("s") * _NC + lax.axis_index("c")
        base = wid * per_w

        def body(t, carry):
            _gather_chunk(table_hbm, idx_hbm, out_hbm, idx_v, rows_v, sem,
                          base + t * 1024, [128] * 8)
            return carry
        lax.fori_loop(0, nch, body, 0)
        if tail:
            _gather_chunk(table_hbm, idx_hbm, out_hbm, idx_v, rows_v, sem,
                          base + nch * 1024, tail_sizes)

    return k(table, idx)


def _scatter_chunk(msg_hbm, dst_hbm, table, idxr_v, idx2_v, msg_v, nbase,
                   off, ngroups, nbatch):
    total = ngroups * 16
    pltpu.sync_copy(dst_hbm.at[pl.ds(off, total)], idxr_v.at[pl.ds(0, total)])
    pltpu.sync_copy(msg_hbm.at[pl.ds(off, total)], msg_v.at[pl.ds(0, total)])
    for g in range(ngroups):
        raw = idxr_v[pl.ds(g * 16, 16)]
        li = raw - nbase
        oob = (li < 0) | (li >= _NROW)
        li = jnp.where(oob, _NROW + (raw & (_NGARB - 1)), li)
        idx2_v[g // 8, pl.ds((g % 8) * 16, 16)] = li
    garb = _NROW + jnp.arange(16, dtype=jnp.int32)
    for g in range(ngroups, nbatch * 8):
        idx2_v[g // 8, pl.ds((g % 8) * 16, 16)] = garb
    for j in range(nbatch):
        pltpu.sync_copy(msg_v.at[pl.ds(j * 128, 128)],
                        table.at[idx2_v.at[j]], add=True)


def _sc_scatter(msg, dst):
    e, dd = msg.shape
    per_w = e // _NS
    chunk = 256
    nch = per_w // chunk
    tail = per_w - nch * chunk
    tail_groups = tail // 16
    tail_batch = (tail + 127) // 128
    zrows = _NROW // _NS
    zeros = jnp.zeros((zrows, dd), jnp.float32)

    @functools.partial(
        pl.kernel, mesh=_mesh(),
        out_type=jax.ShapeDtypeStruct((_NPAD, dd), jnp.float32),
        scratch_types=[
            pltpu.VMEM_SHARED((_NTAB, dd), jnp.float32),
            pltpu.VMEM((chunk,), jnp.int32),
            pltpu.VMEM((chunk // 128, 128), jnp.int32),
            pltpu.VMEM((chunk, dd), jnp.float32),
        ],
        compiler_params=_SC_PARAMS,
        name="sc_scatter%d" % dd)
    def k(msg_hbm, dst_hbm, zeros_hbm, out_hbm, table, idxr_v, idx2_v, msg_v):
        cid = lax.axis_index("c")
        sid = lax.axis_index("s")
        nbase = cid * _NROW
        pltpu.sync_copy(zeros_hbm, table.at[pl.ds(sid * zrows, zrows)])
        plsc.subcore_barrier()
        ebase = sid * per_w

        def body(t, carry):
            _scatter_chunk(msg_hbm, dst_hbm, table, idxr_v, idx2_v, msg_v,
                           nbase, ebase + t * chunk, chunk // 16, chunk // 128)
            return carry
        lax.fori_loop(0, nch, body, 0)
        if tail:
            _scatter_chunk(msg_hbm, dst_hbm, table, idxr_v, idx2_v, msg_v,
                           nbase, ebase + nch * chunk, tail_groups, tail_batch)
        plsc.subcore_barrier()
        pltpu.sync_copy(table.at[pl.ds(sid * zrows, zrows)],
                        out_hbm.at[pl.ds(nbase + sid * zrows, zrows)])

    return k(msg, dst, zeros)


def _sc_count(dst):
    e = dst.shape[0]
    per_w = e // _NS
    nch = per_w // 1024
    tail = per_w - nch * 1024
    tail_groups = tail // 16
    tail_batch = (tail + 127) // 128
    zrows = _NROW // _NS
    zeros = jnp.zeros((zrows,), jnp.float32)

    @functools.partial(
        pl.kernel, mesh=_mesh(),
        out_type=jax.ShapeDtypeStruct((_NPAD,), jnp.float32),
        scratch_types=[
            pltpu.VMEM_SHARED((_NTAB,), jnp.float32),
            pltpu.VMEM((1024,), jnp.int32),
            pltpu.VMEM((8, 128), jnp.int32),
            pltpu.VMEM((128,), jnp.float32),
        ],
        compiler_params=_SC_PARAMS,
        name="sc_count")
    def k(dst_hbm, zeros_hbm, out_hbm, table, idxr_v, idx2_v, ones_v):
        cid = lax.axis_index("c")
        sid = lax.axis_index("s")
        nbase = cid * _NROW
        pltpu.sync_copy(zeros_hbm, table.at[pl.ds(sid * zrows, zrows)])
        for i in range(8):
            ones_v[pl.ds(i * 16, 16)] = jnp.ones((16,), jnp.float32)
        plsc.subcore_barrier()
        ebase = sid * per_w

        def chunk(off, ngroups, nbatch):
            total = ngroups * 16
            pltpu.sync_copy(dst_hbm.at[pl.ds(off, total)],
                            idxr_v.at[pl.ds(0, total)])
            for g in range(ngroups):
                raw = idxr_v[pl.ds(g * 16, 16)]
                li = raw - nbase
                oob = (li < 0) | (li >= _NROW)
                li = jnp.where(oob, _NROW + (raw & (_NGARB - 1)), li)
                idx2_v[g // 8, pl.ds((g % 8) * 16, 16)] = li
            garb = _NROW + jnp.arange(16, dtype=jnp.int32)
            for g in range(ngroups, nbatch * 8):
                idx2_v[g // 8, pl.ds((g % 8) * 16, 16)] = garb
            for j in range(nbatch):
                pltpu.sync_copy(ones_v, table.at[idx2_v.at[j]], add=True)

        def body(t, carry):
            chunk(ebase + t * 1024, 64, 8)
            return carry
        lax.fori_loop(0, nch, body, 0)
        if tail:
            chunk(ebase + nch * 1024, tail_groups, tail_batch)
        plsc.subcore_barrier()
        pltpu.sync_copy(table.at[pl.ds(sid * zrows, zrows)],
                        out_hbm.at[pl.ds(nbase + sid * zrows, zrows)])

    return k(dst, zeros)


_gather_impl = _sc_gather
_scatter_impl = _sc_scatter
_count_impl = _sc_count


def _layer_weights(p):
    w1 = p["w1"]
    w1a, w1b, w1c = w1[:SDIM], w1[SDIM:2 * SDIM], w1[2 * SDIM:]
    w2 = p["w2"]
    b2 = p["b2"]
    mlp_wts = [w1c, w2[:, :SDIM], b2[None, :SDIM],
               w2[:, SDIM:SDIM + VDIM], b2[None, SDIM:SDIM + VDIM],
               w2[:, SDIM + VDIM:], b2[None, SDIM + VDIM:]]
    prep_wts = [p["ln_gamma"][None], p["ln_beta"][None], w1a, w1b,
                p["b1"][None]]
    return mlp_wts, prep_wts


def kernel(s, v, edge_index_local, d_local, vec_local, edge_index_global,
           d_global, vec_global, params):
    n = s.shape[0]
    v2 = v.reshape(n, V3)
    src_l, dst_l = edge_index_local[0], edge_index_local[1]
    src_g, dst_g = edge_index_global[0], edge_index_global[1]

    cnt_l = _count_impl(dst_l)[:n, None]
    cnt_g = _count_impl(dst_g)[:n, None]

    pg = params["layer%d" % (NUM_LAYERS - 2)]
    x_l, dir_l = _edge_static(True, d_local[:, None], vec_local, [])
    x_g, dir_g = _edge_static(False, d_global[:, None], vec_global,
                              [pg["w_d"], pg["b_d"][None]])

    p0 = params["layer0"]
    mlp_wts, prep_wts = _layer_weights(p0)
    sln, vln, a, b = _node_prep(s, v2, *prep_wts)

    for i in range(NUM_LAYERS):
        p = params["layer%d" % i]
        use_global = i == NUM_LAYERS - 2
        has_v = i > 0
        src, dst = (src_g, dst_g) if use_global else (src_l, dst_l)
        x, dir48 = (x_g, dir_g) if use_global else (x_l, dir_l)
        cnt = cnt_g if use_global else cnt_l

        if use_global:
            ww = mlp_wts[0]
        else:
            ww = jnp.dot(p["w_rbf"], mlp_wts[0])
        ewts = [ww] + mlp_wts[1:]

        asrc = _gather_impl(a, src)
        bdst = _gather_impl(b, dst)
        vsrc = _gather_impl(vln, src) if has_v else None

        smsg, vmsg = _edge_mlp(has_v, asrc, bdst, x, dir48, vsrc, ewts)

        sagg = _scatter_impl(smsg, dst)
        vagg = _scatter_impl(vmsg, dst)

        if i < NUM_LAYERS - 1:
            uwts = [p["wu1"], p["bu1"][None], p["wu2"], p["bu2"][None]]
            mlp_wts, prep_wts = _layer_weights(params["layer%d" % (i + 1)])
            sln, vln, a, b = _node_update_prep(
                sln, vln, sagg[:n], vagg[:n], cnt, uwts, prep_wts)
        else:
            s_out, v_out = _node_update(False, sln, vln, sagg[:n], vagg[:n],
                                        cnt, [])

    return s_out, v_out.reshape(n, 3, VDIM)
